# initial kernel scaffold (unmeasured)
import jax
import jax.numpy as jnp
from jax import lax
from jax.experimental import pallas as pl
from jax.experimental.pallas import tpu as pltpu

N_DEV = 4


def kernel(O, Wo):
    B, S, Hs, D = O.shape
    K = Hs * D
    F = Wo.shape[1]
    s_chunk = S // N_DEV

    P = (
        lax.dot_general(
            O.astype(jnp.bfloat16).reshape(B * S, K),
            Wo.astype(jnp.bfloat16),
            (((1,), (0,)), ((), ())),
            preferred_element_type=jnp.float32,
        )
        .astype(jnp.bfloat16)
        .reshape(B, S, F)
    )

    def body(p_hbm, out_ref, comm_ref, stage_ref, send_sems, recv_sems, copy_sem):
        me = lax.axis_index("i")
        left = (me - 1) % N_DEV
        right = (me + 1) % N_DEV

        barrier_sem = pltpu.get_barrier_semaphore()
        for nbr in [left, right]:
            pl.semaphore_signal(
                barrier_sem, inc=1,
                device_id=(nbr,), device_id_type=pl.DeviceIdType.MESH,
            )
        pl.semaphore_wait(barrier_sem, 2)

        def load_chunk(c, dst):
            cp = pltpu.make_async_copy(
                p_hbm.at[:, pl.ds(c * s_chunk, s_chunk), :], dst, copy_sem
            )
            cp.start()
            cp.wait()

        load_chunk((me - 1) % N_DEV, comm_ref.at[0])

        for h in range(N_DEV - 1):
            rdma = pltpu.make_async_remote_copy(
                src_ref=comm_ref.at[h],
                dst_ref=comm_ref.at[h + 1],
                send_sem=send_sems.at[h],
                recv_sem=recv_sems.at[h + 1],
                device_id=(right,),
                device_id_type=pl.DeviceIdType.MESH,
            )
            rdma.start()
            load_chunk((me - 2 - h) % N_DEV, stage_ref)
            rdma.wait()
            if h < N_DEV - 2:
                comm_ref[h + 1] = comm_ref[h + 1] + stage_ref[...]
            else:
                out_ref[...] = comm_ref[h + 1] + stage_ref[...]

    out_bf16 = pl.pallas_call(
        body,
        out_shape=jax.ShapeDtypeStruct((B, s_chunk, F), jnp.bfloat16),
        in_specs=[pl.BlockSpec(memory_space=pltpu.ANY)],
        out_specs=pl.BlockSpec(memory_space=pltpu.VMEM),
        scratch_shapes=[
            pltpu.VMEM((N_DEV, B, s_chunk, F), jnp.bfloat16),
            pltpu.VMEM((B, s_chunk, F), jnp.bfloat16),
            pltpu.SemaphoreType.DMA((N_DEV,)),
            pltpu.SemaphoreType.DMA((N_DEV,)),
            pltpu.SemaphoreType.DMA,
        ],
        compiler_params=pltpu.CompilerParams(collective_id=0),
    )(P)
    return out_bf16.astype(jnp.float32)


# baseline (device time: 368539 ns/iter reference)
import jax
import jax.numpy as jnp
from jax import lax
from jax.experimental import pallas as pl
from jax.experimental.pallas import tpu as pltpu

N_DEV = 4


def kernel(O, Wo):
    B, S, Hs, D = O.shape
    K = Hs * D
    F = Wo.shape[1]
    s_chunk = S // N_DEV

    P = (
        lax.dot_general(
            O.astype(jnp.bfloat16).reshape(B * S, K),
            Wo.astype(jnp.bfloat16),
            (((1,), (0,)), ((), ())),
            preferred_element_type=jnp.float32,
        )
        .astype(jnp.bfloat16)
        .reshape(B, S, F)
    )

    def body(p_hbm, out_ref, comm_ref, stage_ref, send_sems, recv_sems, copy_sem):
        me = lax.axis_index("i")
        left = (me - 1) % N_DEV
        right = (me + 1) % N_DEV

        barrier_sem = pltpu.get_barrier_semaphore()
        for nbr in [left, right]:
            pl.semaphore_signal(
                barrier_sem, inc=1,
                device_id=(nbr,), device_id_type=pl.DeviceIdType.MESH,
            )
        pl.semaphore_wait(barrier_sem, 2)

        def load_chunk(c, dst):
            cp = pltpu.make_async_copy(
                p_hbm.at[:, pl.ds(c * s_chunk, s_chunk), :], dst, copy_sem
            )
            cp.start()
            cp.wait()

        load_chunk((me - 1) % N_DEV, comm_ref.at[0])

        for h in range(N_DEV - 1):
            rdma = pltpu.make_async_remote_copy(
                src_ref=comm_ref.at[h],
                dst_ref=comm_ref.at[h + 1],
                send_sem=send_sems.at[h],
                recv_sem=recv_sems.at[h + 1],
                device_id=(right,),
                device_id_type=pl.DeviceIdType.MESH,
            )
            rdma.start()
            load_chunk((me - 2 - h) % N_DEV, stage_ref)
            rdma.wait()
            if h < N_DEV - 2:
                comm_ref[h + 1] = comm_ref[h + 1] + stage_ref[...]
            else:
                out_ref[...] = comm_ref[h + 1] + stage_ref[...]

    out_bf16 = pl.pallas_call(
        body,
        out_shape=jax.ShapeDtypeStruct((B, s_chunk, F), jnp.bfloat16),
        in_specs=[pl.BlockSpec(memory_space=pl.ANY)],
        out_specs=pl.BlockSpec(memory_space=pltpu.VMEM),
        scratch_shapes=[
            pltpu.VMEM((N_DEV, B, s_chunk, F), jnp.bfloat16),
            pltpu.VMEM((B, s_chunk, F), jnp.bfloat16),
            pltpu.SemaphoreType.DMA((N_DEV,)),
            pltpu.SemaphoreType.DMA((N_DEV,)),
            pltpu.SemaphoreType.DMA,
        ],
        compiler_params=pltpu.CompilerParams(
            collective_id=0, vmem_limit_bytes=100 * 1024 * 1024
        ),
    )(P)
    return out_bf16.astype(jnp.float32)


# device time: 330249 ns/iter; 1.1159x vs baseline; 1.1159x over previous
import jax
import jax.numpy as jnp
from jax import lax
from jax.experimental import pallas as pl
from jax.experimental.pallas import tpu as pltpu

N_DEV = 4


def kernel(O, Wo):
    B, S, Hs, D = O.shape
    K = Hs * D
    F = Wo.shape[1]
    s_chunk = S // N_DEV
    n_tile = 2
    s_tile = s_chunk // n_tile

    O3 = O.astype(jnp.bfloat16).reshape(B, S, K)
    Wo16 = Wo.astype(jnp.bfloat16)

    def body(o_hbm, wo_ref, out_ref, comm_ref, o_stage, pc_ref,
             send_sems, recv_sems, o_sems, out_sem):
        me = lax.axis_index("i")
        left = (me - 1) % N_DEV
        right = (me + 1) % N_DEV

        barrier_sem = pltpu.get_barrier_semaphore()
        for nbr in [left, right]:
            pl.semaphore_signal(
                barrier_sem, inc=1,
                device_id=(nbr,), device_id_type=pl.DeviceIdType.MESH,
            )
        pl.semaphore_wait(barrier_sem, 2)

        def start_o_load(c, buf):
            cp = pltpu.make_async_copy(
                o_hbm.at[:, pl.ds(c * s_chunk, s_chunk), :],
                o_stage.at[buf],
                o_sems.at[buf],
            )
            cp.start()
            return cp

        def compute_pc(buf, dst):
            for b in range(B):
                for t in range(n_tile):
                    rows = pl.ds(t * s_tile, s_tile)
                    dst[b, rows, :] = jnp.dot(
                        o_stage[buf, b, t * s_tile:(t + 1) * s_tile, :],
                        wo_ref[...],
                        preferred_element_type=jnp.float32,
                    ).astype(jnp.bfloat16)

        cp = start_o_load((me - 1) % N_DEV, 0)
        cp.wait()
        compute_pc(0, comm_ref.at[0])
        pending = start_o_load((me - 2) % N_DEV, 1)

        for h in range(N_DEV - 1):
            rdma = pltpu.make_async_remote_copy(
                src_ref=comm_ref.at[h],
                dst_ref=comm_ref.at[h + 1],
                send_sem=send_sems.at[h],
                recv_sem=recv_sems.at[h + 1],
                device_id=(right,),
                device_id_type=pl.DeviceIdType.MESH,
            )
            rdma.start()
            buf = (h + 1) % 2
            pending.wait()
            compute_pc(buf, pc_ref)
            if h < N_DEV - 2:
                pending = start_o_load((me - 3 - h) % N_DEV, (buf + 1) % 2)
            rdma.wait()
            comm_ref[h + 1] = comm_ref[h + 1] + pc_ref[...]

        out_cp = pltpu.make_async_copy(comm_ref.at[N_DEV - 1], out_ref, out_sem)
        out_cp.start()
        out_cp.wait()

    out_bf16 = pl.pallas_call(
        body,
        out_shape=jax.ShapeDtypeStruct((B, s_chunk, F), jnp.bfloat16),
        in_specs=[
            pl.BlockSpec(memory_space=pl.ANY),
            pl.BlockSpec(memory_space=pltpu.VMEM),
        ],
        out_specs=pl.BlockSpec(memory_space=pl.ANY),
        scratch_shapes=[
            pltpu.VMEM((N_DEV, B, s_chunk, F), jnp.bfloat16),
            pltpu.VMEM((2, B, s_chunk, K), jnp.bfloat16),
            pltpu.VMEM((B, s_chunk, F), jnp.bfloat16),
            pltpu.SemaphoreType.DMA((N_DEV,)),
            pltpu.SemaphoreType.DMA((N_DEV,)),
            pltpu.SemaphoreType.DMA((2,)),
            pltpu.SemaphoreType.DMA,
        ],
        compiler_params=pltpu.CompilerParams(
            collective_id=0, vmem_limit_bytes=100 * 1024 * 1024
        ),
    )(O3, Wo16)
    return out_bf16.astype(jnp.float32)


# device time: 319563 ns/iter; 1.1533x vs baseline; 1.0334x over previous
import jax
import jax.numpy as jnp
from jax import lax
from jax.experimental import pallas as pl
from jax.experimental.pallas import tpu as pltpu

N_DEV = 4
N_HALF = 2


def kernel(O, Wo):
    B, S, Hs, D = O.shape
    K = Hs * D
    F = Wo.shape[1]
    s_chunk = S // N_DEV
    s_half = s_chunk // N_HALF

    O3 = O.astype(jnp.bfloat16).reshape(B, S, K)
    Wo16 = Wo.astype(jnp.bfloat16)

    def body(o_hbm, wo_ref, out_ref, comm_ref, o_stage, pc_ref,
             send_sems, recv_sems, o_sems, out_sem):
        me = lax.axis_index("i")
        left = (me - 1) % N_DEV
        right = (me + 1) % N_DEV

        barrier_sem = pltpu.get_barrier_semaphore()
        for nbr in [left, right]:
            pl.semaphore_signal(
                barrier_sem, inc=1,
                device_id=(nbr,), device_id_type=pl.DeviceIdType.MESH,
            )
        pl.semaphore_wait(barrier_sem, 2)

        def start_o_load(c, buf):
            cp = pltpu.make_async_copy(
                o_hbm.at[:, pl.ds(c * s_chunk, s_chunk), :],
                o_stage.at[buf],
                o_sems.at[buf],
            )
            cp.start()
            return cp

        def half_rows(q):
            return pl.ds(q * s_half, s_half)

        def compute_half(buf, q, dst):
            for b in range(B):
                dst[b, half_rows(q), :] = jnp.dot(
                    o_stage[buf, b, q * s_half:(q + 1) * s_half, :],
                    wo_ref[...],
                    preferred_element_type=jnp.float32,
                ).astype(jnp.bfloat16)

        def make_rdma(h, q):
            return pltpu.make_async_remote_copy(
                src_ref=comm_ref.at[h, :, half_rows(q), :],
                dst_ref=comm_ref.at[h + 1, :, half_rows(q), :],
                send_sem=send_sems.at[h, q],
                recv_sem=recv_sems.at[h + 1, q],
                device_id=(right,),
                device_id_type=pl.DeviceIdType.MESH,
            )

        cp = start_o_load((me - 1) % N_DEV, 0)
        cp.wait()
        all_rdmas = []
        hop_rdmas = []
        for q in range(N_HALF):
            compute_half(0, q, comm_ref.at[0])
            r = make_rdma(0, q)
            r.start()
            hop_rdmas.append(r)
            all_rdmas.append(r)
        pending = start_o_load((me - 2) % N_DEV, 1)

        for h in range(N_DEV - 1):
            buf = (h + 1) % 2
            pending.wait()
            for q in range(N_HALF):
                compute_half(buf, q, pc_ref)
            if h < N_DEV - 2:
                pending = start_o_load((me - 3 - h) % N_DEV, (buf + 1) % 2)
            next_rdmas = []
            for q in range(N_HALF):
                hop_rdmas[q].wait_recv()
                rows = half_rows(q)
                comm_ref[h + 1, :, rows, :] = (
                    comm_ref[h + 1, :, rows, :] + pc_ref[:, rows, :]
                )
                if h < N_DEV - 2:
                    r = make_rdma(h + 1, q)
                    r.start()
                    next_rdmas.append(r)
                    all_rdmas.append(r)
            hop_rdmas = next_rdmas

        for r in all_rdmas:
            r.wait_send()

        out_cp = pltpu.make_async_copy(comm_ref.at[N_DEV - 1], out_ref, out_sem)
        out_cp.start()
        out_cp.wait()

    out_bf16 = pl.pallas_call(
        body,
        out_shape=jax.ShapeDtypeStruct((B, s_chunk, F), jnp.bfloat16),
        in_specs=[
            pl.BlockSpec(memory_space=pl.ANY),
            pl.BlockSpec(memory_space=pltpu.VMEM),
        ],
        out_specs=pl.BlockSpec(memory_space=pl.ANY),
        scratch_shapes=[
            pltpu.VMEM((N_DEV, B, s_chunk, F), jnp.bfloat16),
            pltpu.VMEM((2, B, s_chunk, K), jnp.bfloat16),
            pltpu.VMEM((B, s_chunk, F), jnp.bfloat16),
            pltpu.SemaphoreType.DMA((N_DEV, N_HALF)),
            pltpu.SemaphoreType.DMA((N_DEV, N_HALF)),
            pltpu.SemaphoreType.DMA((2,)),
            pltpu.SemaphoreType.DMA,
        ],
        compiler_params=pltpu.CompilerParams(
            collective_id=0, vmem_limit_bytes=100 * 1024 * 1024
        ),
    )(O3, Wo16)
    return out_bf16.astype(jnp.float32)
